# baseline (speedup 1.0000x reference)
import jax
import jax.numpy as jnp
from jax import lax
from jax.experimental import pallas as pl
from jax.experimental.pallas import tpu as pltpu
from jax.experimental.pallas import tpu_sc as plsc

B = 16384
D = 32
NC = 2
NS = 16
NW = NC * NS
BPW = B // NW
EPW = BPW * D


def _gather_body(f0, f1, f2, f3, e0, e1, e2, e3, out_hbm, eidx_v, vals_v, sem):
    wid = lax.axis_index("s") * NC + lax.axis_index("c")
    base = wid * EPW
    for t, (fh, eh) in enumerate(((f0, e0), (f1, e1), (f2, e2), (f3, e3))):
        pltpu.sync_copy(eh.at[pl.ds(base, EPW)], eidx_v)
        pltpu.async_copy(fh.at[eidx_v], vals_v, sem).wait()
        pltpu.sync_copy(vals_v, out_hbm.at[pl.ds((t * B * D) + base, EPW)])


def kernel(room_id, hotel, room_type, room_name,
           room_table, hotel_table, room_type_table, room_name_table):
    mesh = plsc.VectorSubcoreMesh(core_axis_name="c", subcore_axis_name="s")
    gather = pl.kernel(
        _gather_body,
        out_type=jax.ShapeDtypeStruct((4 * B * D,), jnp.float32),
        mesh=mesh,
        scratch_types=[
            pltpu.VMEM((EPW,), jnp.int32),
            pltpu.VMEM((EPW,), jnp.float32),
            pltpu.SemaphoreType.DMA,
        ],
    )
    lane = jnp.arange(D, dtype=jnp.int32)
    NR = room_table.shape[0]
    eidx = [(lane[None, :] * NR + room_id.astype(jnp.int32)[:, None]).reshape(-1)]
    eidx += [
        (idx.astype(jnp.int32)[:, None] * D + lane[None, :]).reshape(-1)
        for idx in (hotel, room_type, room_name)
    ]
    flats = [room_table.T.reshape(-1)]
    flats += [t.reshape(-1) for t in
              (hotel_table, room_type_table, room_name_table)]
    out = gather(*flats, *eidx)
    return out.reshape(4, B, D).transpose(1, 0, 2).reshape(B, 4 * D)


# speedup vs baseline: 2.7611x; 2.7611x over previous
import dataclasses

import jax
import jax.numpy as jnp
from jax import lax
from jax.experimental import pallas as pl
from jax.experimental.pallas import tpu as pltpu
from jax.experimental.pallas import tpu_sc as plsc

B = 16384
D = 32
NC = 2
NS = 16
NW = NC * NS
BPW = B // NW
HALF = BPW // 2
NLANE = 16


def _gather_body(c0, c1, c2, c3, k0h, k1h, k2h, k3h, q0h, q1h, q2h, q3h,
                 out_hbm, ka, kb, qv, rows_v, stage_v, sem):
    wid = lax.axis_index("s") * NC + lax.axis_index("c")
    base = wid * BPW
    iota = lax.broadcasted_iota(jnp.int32, (NLANE,), 0)
    zero = jnp.zeros((NLANE,), jnp.int32)
    for t, (ch, kh, qh) in enumerate(
        ((c0, k0h, q0h), (c1, k1h, q1h), (c2, k2h, q2h), (c3, k3h, q3h))
    ):
        pltpu.sync_copy(kh.at[pl.ds(base, HALF)], ka)
        pltpu.sync_copy(kh.at[pl.ds(base + HALF, HALF)], kb)
        pltpu.sync_copy(qh.at[pl.ds(base, BPW)], qv)
        for h, kref in enumerate((ka, kb)):
            pltpu.sync_copy(ch.at[kref], rows_v)

            @pl.loop(0, HALF // NLANE)
            def _(g):
                ridx = iota + g * NLANE
                rout = ridx + h * HALF
                q16 = qv[pl.ds(h * HALF + g * NLANE, NLANE)]
                cbase = q16 * D
                for c in range(D):
                    val = plsc.load_gather(rows_v, [ridx, cbase + c])
                    plsc.store_scatter(stage_v, [rout, zero + (t * D + c)], val)

    pltpu.sync_copy(stage_v, out_hbm.at[pl.ds(base, BPW)])


def _chunked(t):
    v = t.shape[0]
    pad = (-v) % 4
    if pad:
        t = jnp.pad(t, ((0, pad), (0, 0)))
    return t.reshape(-1, 4 * D)


def kernel(room_id, hotel, room_type, room_name,
           room_table, hotel_table, room_type_table, room_name_table):
    mesh = plsc.VectorSubcoreMesh(core_axis_name="c", subcore_axis_name="s")
    cp = pltpu.CompilerParams()
    if "needs_layout_passes" in pltpu.CompilerParams.__dataclass_fields__:
        cp = dataclasses.replace(cp, needs_layout_passes=False)
    gather = pl.kernel(
        _gather_body,
        out_type=jax.ShapeDtypeStruct((B, 4 * D), jnp.float32),
        mesh=mesh,
        compiler_params=cp,
        scratch_types=[
            pltpu.VMEM((HALF,), jnp.int32),
            pltpu.VMEM((HALF,), jnp.int32),
            pltpu.VMEM((BPW,), jnp.int32),
            pltpu.VMEM((HALF, 4 * D), jnp.float32),
            pltpu.VMEM((BPW, 4 * D), jnp.float32),
            pltpu.SemaphoreType.DMA,
        ],
    )
    chunks = [_chunked(t) for t in
              (room_table, hotel_table, room_type_table, room_name_table)]
    idxs = [i.astype(jnp.int32)
            for i in (room_id, hotel, room_type, room_name)]
    ks = [i >> 2 for i in idxs]
    qs = [i & 3 for i in idxs]
    return gather(*chunks, *ks, *qs)


# speedup vs baseline: 4.2095x; 1.5246x over previous
import dataclasses

import jax
import jax.numpy as jnp
from jax import lax
from jax.experimental import pallas as pl
from jax.experimental.pallas import tpu as pltpu
from jax.experimental.pallas import tpu_sc as plsc

B = 16384
D = 32
NC = 2
NS = 16
NW = NC * NS
BPW = B // NW
HALF = BPW // 2
NLANE = 16


def _gather_body(c0, c1, c2, c3, k0h, k1h, k2h, k3h, q0h, q1h, q2h, q3h,
                 out_hbm, ka, kb, qv, rows_v, stage_v, sem):
    wid = lax.axis_index("s") * NC + lax.axis_index("c")
    base = wid * BPW
    iota = lax.broadcasted_iota(jnp.int32, (NLANE,), 0)
    zero = jnp.zeros((NLANE,), jnp.int32)
    for t, (ch, kh, qh) in enumerate(
        ((c0, k0h, q0h), (c1, k1h, q1h), (c2, k2h, q2h), (c3, k3h, q3h))
    ):
        pltpu.sync_copy(kh.at[pl.ds(base, HALF)], ka)
        pltpu.sync_copy(kh.at[pl.ds(base + HALF, HALF)], kb)
        pltpu.sync_copy(qh.at[pl.ds(base, BPW)], qv)
        for h, kref in enumerate((ka, kb)):
            pltpu.sync_copy(ch.at[kref], rows_v)

            @pl.loop(0, HALF // NLANE)
            def _(g):
                ridx = iota + g * NLANE
                rout = ridx + h * HALF
                q16 = qv[pl.ds(h * HALF + g * NLANE, NLANE)]
                cbase = q16 * D
                for c in range(D):
                    val = plsc.load_gather(rows_v, [ridx, cbase + c])
                    plsc.store_scatter(stage_v, [rout, zero + (t * D + c)], val)

    pltpu.sync_copy(stage_v, out_hbm.at[pl.ds(base, BPW)])


def _chunked(t):
    v4 = (t.shape[0] // 4) * 4
    return t[:v4].reshape(-1, 4 * D)


def kernel(room_id, hotel, room_type, room_name,
           room_table, hotel_table, room_type_table, room_name_table):
    mesh = plsc.VectorSubcoreMesh(core_axis_name="c", subcore_axis_name="s")
    cp = pltpu.CompilerParams()
    if "needs_layout_passes" in pltpu.CompilerParams.__dataclass_fields__:
        cp = dataclasses.replace(cp, needs_layout_passes=False)
    gather = pl.kernel(
        _gather_body,
        out_type=jax.ShapeDtypeStruct((B, 4 * D), jnp.float32),
        mesh=mesh,
        compiler_params=cp,
        scratch_types=[
            pltpu.VMEM((HALF,), jnp.int32),
            pltpu.VMEM((HALF,), jnp.int32),
            pltpu.VMEM((BPW,), jnp.int32),
            pltpu.VMEM((HALF, 4 * D), jnp.float32),
            pltpu.VMEM((BPW, 4 * D), jnp.float32),
            pltpu.SemaphoreType.DMA,
        ],
    )
    tables = (room_table, hotel_table, room_type_table, room_name_table)
    chunks = [_chunked(t) for t in tables]
    idxs = [i.astype(jnp.int32)
            for i in (room_id, hotel, room_type, room_name)]
    nchunks = [c.shape[0] for c in chunks]
    ks = [jnp.minimum(i >> 2, n - 1) for i, n in zip(idxs, nchunks)]
    qs = [i & 3 for i in idxs]
    out = gather(*chunks, *ks, *qs)
    out3 = out.reshape(B, 4, D)
    idx_stack = jnp.stack(idxs, axis=1)
    v4s = jnp.array([(t.shape[0] // 4) * 4 for t in tables], jnp.int32)
    tails = jnp.stack([t[-1] for t in tables], axis=0)
    need = idx_stack >= v4s[None, :]
    out3 = jnp.where(need[:, :, None], tails[None, :, :], out3)
    return out3.reshape(B, 4 * D)


# speedup vs baseline: 9.6803x; 2.2997x over previous
import dataclasses

import jax
import jax.numpy as jnp
from jax import lax
from jax.experimental import pallas as pl
from jax.experimental.pallas import tpu as pltpu
from jax.experimental.pallas import tpu_sc as plsc

B = 16384
D = 32
NC = 2
NS = 16
NW = NC * NS
BPW = B // NW
HALF = BPW // 2
NLANE = 16


def _gather_body(c0, c1, c2, c3, k0h, k1h, k2h, k3h, q0h, q1h, q2h, q3h,
                 out_hbm, ka, kb, qv, rows_v, stage_v, sem):
    wid = lax.axis_index("s") * NC + lax.axis_index("c")
    base = wid * BPW
    iota = lax.broadcasted_iota(jnp.int32, (NLANE,), 0)
    zero = jnp.zeros((NLANE,), jnp.int32)
    for t, (ch, kh, qh) in enumerate(
        ((c0, k0h, q0h), (c1, k1h, q1h), (c2, k2h, q2h), (c3, k3h, q3h))
    ):
        pltpu.sync_copy(kh.at[pl.ds(base, HALF)], ka)
        pltpu.sync_copy(kh.at[pl.ds(base + HALF, HALF)], kb)
        pltpu.sync_copy(qh.at[pl.ds(base, BPW)], qv)
        for h, kref in enumerate((ka, kb)):
            pltpu.sync_copy(ch.at[kref], rows_v)

            @pl.loop(0, HALF // NLANE)
            def _(g):
                ridx = iota + g * NLANE
                rout = ridx + h * HALF
                q16 = qv[pl.ds(h * HALF + g * NLANE, NLANE)]
                cbase = q16 * D
                for c in range(D):
                    val = plsc.load_gather(rows_v, [ridx, cbase + c])
                    plsc.store_scatter(stage_v, [rout, zero + (t * D + c)], val)

    pltpu.sync_copy(stage_v, out_hbm.at[pl.ds(base, BPW)])


def _detile_body(i0, i1, i2, i3, out_ref):
    out_ref[...] = jnp.concatenate(
        [i0[...], i1[...], i2[...], i3[...]], axis=0
    ).T


KB = 2048


def _chunked(t):
    v = t.shape[0]
    grid = (v + 4 * KB - 1) // (4 * KB)
    v4 = grid * KB
    vblk = (v + KB - 1) // KB
    tt = t.T
    out = pl.pallas_call(
        _detile_body,
        grid=(grid,),
        in_specs=[
            pl.BlockSpec(
                (D, KB),
                lambda j, q=q, g=grid, m=vblk - 1: (0, jnp.minimum(q * g + j, m)),
            )
            for q in range(4)
        ],
        out_specs=pl.BlockSpec((KB, 4 * D), lambda j: (j, 0)),
        out_shape=jax.ShapeDtypeStruct((v4, 4 * D), jnp.float32),
    )(tt, tt, tt, tt)
    return out, v4


def kernel(room_id, hotel, room_type, room_name,
           room_table, hotel_table, room_type_table, room_name_table):
    mesh = plsc.VectorSubcoreMesh(core_axis_name="c", subcore_axis_name="s")
    cp = pltpu.CompilerParams()
    if "needs_layout_passes" in pltpu.CompilerParams.__dataclass_fields__:
        cp = dataclasses.replace(cp, needs_layout_passes=False)
    gather = pl.kernel(
        _gather_body,
        out_type=jax.ShapeDtypeStruct((B, 4 * D), jnp.float32),
        mesh=mesh,
        compiler_params=cp,
        scratch_types=[
            pltpu.VMEM((HALF,), jnp.int32),
            pltpu.VMEM((HALF,), jnp.int32),
            pltpu.VMEM((BPW,), jnp.int32),
            pltpu.VMEM((HALF, 4 * D), jnp.float32),
            pltpu.VMEM((BPW, 4 * D), jnp.float32),
            pltpu.SemaphoreType.DMA,
        ],
    )
    tables = (room_table, hotel_table, room_type_table, room_name_table)
    chunked = [_chunked(t) for t in tables]
    chunks = [c for c, _ in chunked]
    idxs = [i.astype(jnp.int32)
            for i in (room_id, hotel, room_type, room_name)]
    ks = [i % v4 for i, (_, v4) in zip(idxs, chunked)]
    qs = [i // v4 for i, (_, v4) in zip(idxs, chunked)]
    return gather(*chunks, *ks, *qs)


# speedup vs baseline: 12.7636x; 1.3185x over previous
import dataclasses

import jax
import jax.numpy as jnp
from jax import lax
from jax.experimental import pallas as pl
from jax.experimental.pallas import tpu as pltpu
from jax.experimental.pallas import tpu_sc as plsc

B = 16384
D = 32
NC = 2
NS = 16
NW = NC * NS
BPW = B // NW
HALF = BPW // 2
NLANE = 16


def _gather_body(c0, c1, c2, c3, k0h, k1h, k2h, k3h, q0h, q1h, q2h, q3h,
                 out_hbm,
                 kc0, kc1, kc2, kc3, kc4, kc5, kc6, kc7,
                 qc0, qc1, qc2, qc3,
                 rows_a, rows_b, stage_v, sem_a, sem_b):
    wid = lax.axis_index("s") * NC + lax.axis_index("c")
    base = wid * BPW
    iota = lax.broadcasted_iota(jnp.int32, (NLANE,), 0)
    zero = jnp.zeros((NLANE,), jnp.int32)
    tabs = ((c0, k0h, q0h), (c1, k1h, q1h), (c2, k2h, q2h), (c3, k3h, q3h))
    kall = (kc0, kc1, kc2, kc3, kc4, kc5, kc6, kc7)
    qall = (qc0, qc1, qc2, qc3)
    bufs = (rows_a, rows_b)
    sems = (sem_a, sem_b)
    for t, (ch, kh, qh) in enumerate(tabs):
        pltpu.sync_copy(kh.at[pl.ds(base, HALF)], kall[2 * t])
        pltpu.sync_copy(kh.at[pl.ds(base + HALF, HALF)], kall[2 * t + 1])
        pltpu.sync_copy(qh.at[pl.ds(base, BPW)], qall[t])
    copies = [None] * 8
    copies[0] = pltpu.async_copy(tabs[0][0].at[kall[0]], bufs[0], sems[0])
    for i in range(8):
        h, t = divmod(i, 4)
        if i + 1 < 8:
            hn, tn = divmod(i + 1, 4)
            copies[i + 1] = pltpu.async_copy(
                tabs[tn][0].at[kall[2 * tn + hn]], bufs[(i + 1) % 2],
                sems[(i + 1) % 2],
            )
        copies[i].wait()
        rows_v = bufs[i % 2]
        qv = qall[t]

        @pl.loop(0, HALF // NLANE)
        def _(g):
            ridx = iota + g * NLANE
            q16 = qv[pl.ds(h * HALF + g * NLANE, NLANE)]
            cbase = q16 * D
            for c in range(D):
                val = plsc.load_gather(rows_v, [ridx, cbase + c])
                plsc.store_scatter(stage_v, [ridx, zero + (t * D + c)], val)

        if t == 3:
            pltpu.sync_copy(
                stage_v, out_hbm.at[pl.ds(base + h * HALF, HALF)]
            )


def _detile_body(i0, i1, i2, i3, out_ref):
    out_ref[...] = jnp.concatenate(
        [i0[...], i1[...], i2[...], i3[...]], axis=0
    ).T


KB = 8192


def _chunked(t):
    v = t.shape[0]
    grid = (v + 4 * KB - 1) // (4 * KB)
    v4 = grid * KB
    vblk = (v + KB - 1) // KB
    tt = t.T
    out = pl.pallas_call(
        _detile_body,
        grid=(grid,),
        in_specs=[
            pl.BlockSpec(
                (D, KB),
                lambda j, q=q, g=grid, m=vblk - 1: (0, jnp.minimum(q * g + j, m)),
            )
            for q in range(4)
        ],
        out_specs=pl.BlockSpec((KB, 4 * D), lambda j: (j, 0)),
        out_shape=jax.ShapeDtypeStruct((v4, 4 * D), jnp.float32),
    )(tt, tt, tt, tt)
    return out, v4


def kernel(room_id, hotel, room_type, room_name,
           room_table, hotel_table, room_type_table, room_name_table):
    mesh = plsc.VectorSubcoreMesh(core_axis_name="c", subcore_axis_name="s")
    cp = pltpu.CompilerParams()
    if "needs_layout_passes" in pltpu.CompilerParams.__dataclass_fields__:
        cp = dataclasses.replace(cp, needs_layout_passes=False)
    gather = pl.kernel(
        _gather_body,
        out_type=jax.ShapeDtypeStruct((B, 4 * D), jnp.float32),
        mesh=mesh,
        compiler_params=cp,
        scratch_types=(
            [pltpu.VMEM((HALF,), jnp.int32) for _ in range(8)]
            + [pltpu.VMEM((BPW,), jnp.int32) for _ in range(4)]
            + [pltpu.VMEM((HALF, 4 * D), jnp.float32) for _ in range(2)]
            + [pltpu.VMEM((HALF, 4 * D), jnp.float32)]
            + [pltpu.SemaphoreType.DMA, pltpu.SemaphoreType.DMA]
        ),
    )
    tables = (room_table, hotel_table, room_type_table, room_name_table)
    chunked = [_chunked(t) for t in tables]
    chunks = [c for c, _ in chunked]
    idxs = [i.astype(jnp.int32)
            for i in (room_id, hotel, room_type, room_name)]
    ks = [i % v4 for i, (_, v4) in zip(idxs, chunked)]
    qs = [i // v4 for i, (_, v4) in zip(idxs, chunked)]
    return gather(*chunks, *ks, *qs)
